# Initial kernel scaffold; baseline (speedup 1.0000x reference)
import jax
import jax.numpy as jnp
from jax.experimental import pallas as pl


def kernel(h, x, edge_index, W1m, b1m, W2m, b2m, W1h, b1h, W2h, b2h, ln_w, ln_b, centers, gamma):
    raise NotImplementedError("write your pallas kernel here")


# speedup vs baseline: 3.6584x; 3.6584x over previous
import functools

import jax
import jax.numpy as jnp
from jax import lax
from jax.experimental import pallas as pl
from jax.experimental.pallas import tpu as pltpu
from jax.experimental.pallas import tpu_sc as plsc

N = 10000
E = 320000
D = 128
NRBF = 16
NPAD = 10240

NC = 2
NS = 16
NW = NC * NS
EPW = E // NW
CH = 80
NCHUNK = EPW // CH
LANES = 16

BN = 1000
BE = 2000


def _mesh():
    return plsc.VectorSubcoreMesh(core_axis_name="c", subcore_axis_name="s")


def _tables_body(h_ref, wa_ref, wb_ref, a_ref, b_ref):
    h = h_ref[...]
    a_ref[...] = jnp.dot(h, wa_ref[...], preferred_element_type=jnp.float32)
    b_ref[...] = jnp.dot(h, wb_ref[...], preferred_element_type=jnp.float32)


def _make_tables(h, w1m_a, w1m_b):
    return pl.pallas_call(
        _tables_body,
        grid=(N // BN,),
        in_specs=[
            pl.BlockSpec((BN, D), lambda i: (i, 0)),
            pl.BlockSpec((D, D), lambda i: (0, 0)),
            pl.BlockSpec((D, D), lambda i: (0, 0)),
        ],
        out_specs=[
            pl.BlockSpec((BN, D), lambda i: (i, 0)),
            pl.BlockSpec((BN, D), lambda i: (i, 0)),
        ],
        out_shape=[
            jax.ShapeDtypeStruct((N, D), jnp.float32),
            jax.ShapeDtypeStruct((N, D), jnp.float32),
        ],
    )(h, w1m_a, w1m_b)


@functools.lru_cache(maxsize=None)
def _sc_gather_fn():
    @functools.partial(
        pl.kernel,
        mesh=_mesh(),
        compiler_params=pltpu.CompilerParams(needs_layout_passes=False),
        out_type=[
            jax.ShapeDtypeStruct((E, D), jnp.float32),
            jax.ShapeDtypeStruct((E, D), jnp.float32),
            jax.ShapeDtypeStruct((E,), jnp.float32),
        ],
        scratch_types=[
            pltpu.VMEM((CH,), jnp.int32),
            pltpu.VMEM((CH,), jnp.int32),
            pltpu.VMEM((CH, D), jnp.float32),
            pltpu.VMEM((CH, D), jnp.float32),
            pltpu.VMEM((CH,), jnp.float32),
            pltpu.VMEM((N,), jnp.float32),
            pltpu.VMEM((N,), jnp.float32),
            pltpu.SemaphoreType.DMA,
            pltpu.SemaphoreType.DMA,
        ],
    )
    def _sc_gather(a_hbm, b_hbm, src_hbm, dst_hbm, x0_hbm, x1_hbm,
                   oa_hbm, ob_hbm, r2_hbm,
                   idxs_v, idxd_v, rowsa_v, rowsb_v, r2_v, x0_v, x1_v,
                   sema, semb):
        wid = lax.axis_index("s") * NC + lax.axis_index("c")
        pltpu.sync_copy(x0_hbm, x0_v)
        pltpu.sync_copy(x1_hbm, x1_v)

        def chunk(c, carry):
            base = wid * EPW + c * CH
            pltpu.sync_copy(src_hbm.at[pl.ds(base, CH)], idxs_v)
            pltpu.sync_copy(dst_hbm.at[pl.ds(base, CH)], idxd_v)
            ca = pltpu.async_copy(a_hbm.at[idxs_v], rowsa_v, sema)
            cb = pltpu.async_copy(b_hbm.at[idxd_v], rowsb_v, semb)
            for j in range(CH // LANES):
                ivs = idxs_v[pl.ds(j * LANES, LANES)]
                ivd = idxd_v[pl.ds(j * LANES, LANES)]
                d0 = (plsc.load_gather(x0_v, [ivs])
                      - plsc.load_gather(x0_v, [ivd]))
                d1 = (plsc.load_gather(x1_v, [ivs])
                      - plsc.load_gather(x1_v, [ivd]))
                r2_v[pl.ds(j * LANES, LANES)] = d0 * d0 + d1 * d1
            ca.wait()
            cb.wait()
            pltpu.sync_copy(rowsa_v, oa_hbm.at[pl.ds(base, CH)])
            pltpu.sync_copy(rowsb_v, ob_hbm.at[pl.ds(base, CH)])
            pltpu.sync_copy(r2_v, r2_hbm.at[pl.ds(base, CH)])
            return carry

        lax.fori_loop(0, NCHUNK, chunk, 0)

    return _sc_gather


def _edge_body(za_ref, zb_ref, r2_ref, wr_ref, b1_ref, w2_ref, b2_ref,
               cen_ref, gam_ref, out_ref):
    pre = za_ref[...] + zb_ref[...]
    r = jnp.sqrt(r2_ref[...] + 1e-8)
    diff = r - cen_ref[...]
    rbf = jnp.exp(-gam_ref[...] * diff * diff)
    z = pre + jnp.dot(rbf, wr_ref[...],
                      preferred_element_type=jnp.float32) + b1_ref[...]
    s = z * jax.nn.sigmoid(z)
    out_ref[...] = jnp.dot(s, w2_ref[...],
                           preferred_element_type=jnp.float32) + b2_ref[...]


def _make_edges(oa, ob, r2, w1m_r, b1m, w2m, b2m, cen_row, gam_row):
    return pl.pallas_call(
        _edge_body,
        grid=(E // BE,),
        in_specs=[
            pl.BlockSpec((BE, D), lambda i: (i, 0)),
            pl.BlockSpec((BE, D), lambda i: (i, 0)),
            pl.BlockSpec((BE, 1), lambda i: (i, 0)),
            pl.BlockSpec((NRBF, D), lambda i: (0, 0)),
            pl.BlockSpec((1, D), lambda i: (0, 0)),
            pl.BlockSpec((D, D), lambda i: (0, 0)),
            pl.BlockSpec((1, D), lambda i: (0, 0)),
            pl.BlockSpec((1, NRBF), lambda i: (0, 0)),
            pl.BlockSpec((1, NRBF), lambda i: (0, 0)),
        ],
        out_specs=pl.BlockSpec((BE, D), lambda i: (i, 0)),
        out_shape=jax.ShapeDtypeStruct((E, D), jnp.float32),
    )(oa, ob, r2, w1m_r, b1m, w2m, b2m, cen_row, gam_row)


@functools.lru_cache(maxsize=None)
def _sc_scatter_fn():
    @functools.partial(
        pl.kernel,
        mesh=_mesh(),
        out_type=jax.ShapeDtypeStruct((NC, NPAD, D), jnp.float32),
        scratch_types=[
            pltpu.VMEM((CH,), jnp.int32),
            pltpu.VMEM((CH, D), jnp.float32),
            pltpu.VMEM_SHARED((NPAD, D), jnp.float32),
            pltpu.SemaphoreType.DMA,
        ],
    )
    def _sc_scatter(m_hbm, dst_hbm, zero_hbm, out_hbm, idx_v, rows_v,
                    acc_sh, sem):
        cid = lax.axis_index("c")
        sid = lax.axis_index("s")
        wid = sid * NC + cid
        rows_per_sub = NPAD // NS
        sl = pl.ds(sid * rows_per_sub, rows_per_sub)
        pltpu.sync_copy(zero_hbm.at[sl], acc_sh.at[sl])
        plsc.subcore_barrier()

        def chunk(c, carry):
            base = wid * EPW + c * CH
            pltpu.sync_copy(dst_hbm.at[pl.ds(base, CH)], idx_v)
            pltpu.sync_copy(m_hbm.at[pl.ds(base, CH)], rows_v)
            pltpu.sync_copy(rows_v, acc_sh.at[idx_v], add=True)
            return carry

        lax.fori_loop(0, NCHUNK, chunk, 0)
        plsc.subcore_barrier()
        pltpu.sync_copy(acc_sh.at[sl], out_hbm.at[cid, sl])

    return _sc_scatter


def _node_body(h_ref, s0_ref, s1_ref, w1ha_ref, w1hb_ref,
               b1h_ref, w2h_ref, b2h_ref, lnw_ref, lnb_ref, out_ref):
    h = h_ref[...]
    agg = s0_ref[...] + s1_ref[...]
    pre = (jnp.dot(h, w1ha_ref[...], preferred_element_type=jnp.float32)
           + jnp.dot(agg, w1hb_ref[...], preferred_element_type=jnp.float32)
           + b1h_ref[...])
    t = pre * jax.nn.sigmoid(pre)
    h_up = jnp.dot(t, w2h_ref[...],
                   preferred_element_type=jnp.float32) + b2h_ref[...]
    y = h + h_up
    mu = jnp.mean(y, axis=1, keepdims=True)
    var = jnp.mean((y - mu) ** 2, axis=1, keepdims=True)
    out_ref[...] = ((y - mu) * lax.rsqrt(var + 1e-5) * lnw_ref[...]
                    + lnb_ref[...])


def _make_nodes(h, s0, s1, w1h_a, w1h_b, b1h, w2h, b2h, lnw, lnb):
    return pl.pallas_call(
        _node_body,
        grid=(N // BN,),
        in_specs=[
            pl.BlockSpec((BN, D), lambda i: (i, 0)),
            pl.BlockSpec((BN, D), lambda i: (i, 0)),
            pl.BlockSpec((BN, D), lambda i: (i, 0)),
            pl.BlockSpec((D, D), lambda i: (0, 0)),
            pl.BlockSpec((D, D), lambda i: (0, 0)),
            pl.BlockSpec((1, D), lambda i: (0, 0)),
            pl.BlockSpec((D, D), lambda i: (0, 0)),
            pl.BlockSpec((1, D), lambda i: (0, 0)),
            pl.BlockSpec((1, D), lambda i: (0, 0)),
            pl.BlockSpec((1, D), lambda i: (0, 0)),
        ],
        out_specs=pl.BlockSpec((BN, D), lambda i: (i, 0)),
        out_shape=jax.ShapeDtypeStruct((N, D), jnp.float32),
    )(h, s0, s1, w1h_a, w1h_b, b1h, w2h, b2h, lnw, lnb)


def kernel(h, x, edge_index, W1m, b1m, W2m, b2m, W1h, b1h, W2h, b2h,
           ln_w, ln_b, centers, gamma):
    src = edge_index[0].astype(jnp.int32)
    dst = edge_index[1].astype(jnp.int32)
    x0 = x[:, 0]
    x1 = x[:, 1]
    cen_row = centers[None, :]
    gam_row = jnp.full((1, NRBF), gamma, jnp.float32)
    zero_rows = jnp.zeros((NPAD, D), jnp.float32)

    a_tab, b_tab = _make_tables(h, W1m[:D], W1m[D:2 * D])
    oa, ob, r2 = _sc_gather_fn()(a_tab, b_tab, src, dst, x0, x1)
    m = _make_edges(oa, ob, r2[:, None], W1m[2 * D:], b1m[None, :],
                    W2m, b2m[None, :], cen_row, gam_row)
    s_part = _sc_scatter_fn()(m, dst, zero_rows)
    out = _make_nodes(h, s_part[0, :N], s_part[1, :N], W1h[:D], W1h[D:],
                      b1h[None, :], W2h, b2h[None, :], ln_w[None, :],
                      ln_b[None, :])
    return out
